# baseline (speedup 1.0000x reference)
import functools
import math

import jax
import jax.numpy as jnp
from jax import lax
from jax.experimental import pallas as pl
from jax.experimental.pallas import tpu as pltpu
from jax.experimental.pallas import tpu_sc as plsc

VOCAB = 100000
EMB = 300
DM = 512
DM2 = DM // 2
B = 1024
L = 200
N_TOK = B * L
SCALE = math.sqrt(DM)
G = 4
N_CHUNK = N_TOK // G


BM = 2048


def _bf16_round_bits(y):
    b = lax.bitcast_convert_type(y, jnp.int32)
    lsb = lax.shift_right_logical(b, 16) & jnp.int32(1)
    return b + jnp.int32(0x7FFF) + lsb


def _proj_body(wvt_ref, w_ref, out_ref):
    y = lax.dot_general(
        wvt_ref[...], w_ref[...],
        dimension_numbers=(((0,), (1,)), ((), ())),
        preferred_element_type=jnp.float32,
    )
    lo = lax.shift_right_logical(_bf16_round_bits(y[:, :DM2]), 16)
    hi = _bf16_round_bits(y[:, DM2:]) & jnp.int32(-65536)
    out_ref[...] = hi | lo


def _project_table(word_vectors, W):
    wvt = word_vectors.T
    return pl.pallas_call(
        _proj_body,
        grid=((VOCAB + BM - 1) // BM,),
        in_specs=[
            pl.BlockSpec((EMB, BM), lambda i: (0, i)),
            pl.BlockSpec((DM, EMB), lambda i: (0, 0)),
        ],
        out_specs=pl.BlockSpec((BM, DM2), lambda i: (i, 0)),
        out_shape=jax.ShapeDtypeStruct((VOCAB, DM2), jnp.int32),
    )(wvt, W * SCALE)


_INFO = plsc.get_sparse_core_info()
NC = _INFO.num_cores
NS = _INFO.num_subcores
NW = NC * NS
B_PER_W = N_CHUNK // NW
CHUNK = 80
NITER = B_PER_W // CHUNK
NBUF = 4
LAG = 2


def _gather_sc(table, idx):
    mesh = plsc.VectorSubcoreMesh(core_axis_name="c", subcore_axis_name="s")

    @functools.partial(
        pl.kernel,
        mesh=mesh,
        out_type=jax.ShapeDtypeStruct((N_CHUNK, DM2), jnp.int32),
        scratch_types=[
            pltpu.VMEM((B_PER_W,), jnp.int32),
            pltpu.VMEM((NBUF, CHUNK, DM2), jnp.int32),
        ]
        + [pltpu.SemaphoreType.DMA] * (2 * NBUF),
    )
    def k(table_hbm, idx_hbm, out_hbm, idx_v, raw_v, *sems):
        gsems, wsems = sems[:NBUF], sems[NBUF:]
        wid = lax.axis_index("s") * NC + lax.axis_index("c")
        base = wid * B_PER_W
        pltpu.sync_copy(idx_hbm.at[pl.ds(base, B_PER_W)], idx_v)

        def start_gather(i, buf):
            pltpu.async_copy(
                table_hbm.at[idx_v.at[pl.ds(i * CHUNK, CHUNK)]],
                raw_v.at[buf],
                gsems[buf],
            )

        def wait_gather(buf):
            pltpu.make_async_copy(
                table_hbm.at[idx_v.at[pl.ds(0, CHUNK)]],
                raw_v.at[buf],
                gsems[buf],
            ).wait()

        def start_write(i, buf):
            pltpu.async_copy(
                raw_v.at[buf],
                out_hbm.at[pl.ds(base + i * CHUNK, CHUNK)],
                wsems[buf],
            )

        def wait_write(buf):
            pltpu.make_async_copy(
                raw_v.at[buf],
                out_hbm.at[pl.ds(base, CHUNK)],
                wsems[buf],
            ).wait()

        for b in range(LAG):
            start_gather(b, b)

        def body(j, _):
            for b in range(NBUF):
                i = j * NBUF + b
                wait_gather(b)
                start_write(i, b)
                nxt = i + LAG

                @pl.when(nxt >= NBUF)
                def _():
                    wait_write((b + LAG) % NBUF)

                @pl.when(nxt < NITER)
                def _():
                    start_gather(nxt, (b + LAG) % NBUF)
            return 0

        lax.fori_loop(0, NITER // NBUF, body, 0)
        for b in range(LAG):
            wait_write((NITER - LAG + b) % NBUF)

    return k(table, idx)


BT = 2048


def _widen_body(acc_ref, packed_ref, out_ref):
    del acc_ref
    v = packed_ref[...]
    out_ref[:, :DM2] = v << 16
    out_ref[:, DM2:] = v & jnp.int32(-65536)


def _widen_chunk(acc, packed, g):
    row0 = g * (N_CHUNK // BT)
    return pl.pallas_call(
        _widen_body,
        grid=(N_CHUNK // BT,),
        in_specs=[
            pl.BlockSpec(memory_space=pl.ANY),
            pl.BlockSpec((BT, DM2), lambda i: (i + g * (N_CHUNK // BT), 0)),
        ],
        out_specs=pl.BlockSpec((BT, DM), lambda i: (i + row0, 0)),
        out_shape=jax.ShapeDtypeStruct((N_TOK, DM), jnp.int32),
        input_output_aliases={0: 0},
    )(acc, packed)


def kernel(x, word_vectors, W):
    packed_table = _project_table(word_vectors, W)
    flat_idx = x.reshape(-1)
    parts = [
        _gather_sc(packed_table, flat_idx[g * N_CHUNK:(g + 1) * N_CHUNK])
        for g in range(G)
    ]
    acc = _widen_first(parts[0])
    for g in range(1, G):
        acc = _widen_chunk(acc, parts[g], g)
    out = lax.bitcast_convert_type(acc, jnp.float32)
    return out.reshape(B, L, DM)


def _widen_first(packed):
    return pl.pallas_call(
        _widen_body_first,
        grid=(N_CHUNK // BT,),
        in_specs=[
            pl.BlockSpec((BT, DM2), lambda i: (i, 0)),
        ],
        out_specs=pl.BlockSpec((BT, DM), lambda i: (i, 0)),
        out_shape=jax.ShapeDtypeStruct((N_TOK, DM), jnp.int32),
    )(packed)


def _widen_body_first(packed_ref, out_ref):
    v = packed_ref[...]
    out_ref[:, :DM2] = v << 16
    out_ref[:, DM2:] = v & jnp.int32(-65536)


# speedup vs baseline: 1.7027x; 1.7027x over previous
import functools
import math

import jax
import jax.numpy as jnp
from jax import lax
from jax.experimental import pallas as pl
from jax.experimental.pallas import tpu as pltpu
from jax.experimental.pallas import tpu_sc as plsc

VOCAB = 100000
EMB = 300
DM = 512
B = 1024
L = 200
N_TOK = B * L
SCALE = math.sqrt(DM)


BM = 2048


def _proj_body(wvt_ref, w_ref, out_ref):
    out_ref[...] = lax.dot_general(
        wvt_ref[...], w_ref[...],
        dimension_numbers=(((0,), (1,)), ((), ())),
        preferred_element_type=jnp.float32,
    )


def _project_table(word_vectors, W):
    wvt = word_vectors.T
    return pl.pallas_call(
        _proj_body,
        grid=((VOCAB + BM - 1) // BM,),
        in_specs=[
            pl.BlockSpec((EMB, BM), lambda i: (0, i)),
            pl.BlockSpec((DM, EMB), lambda i: (0, 0)),
        ],
        out_specs=pl.BlockSpec((BM, DM), lambda i: (i, 0)),
        out_shape=jax.ShapeDtypeStruct((VOCAB, DM), jnp.float32),
    )(wvt, W * SCALE)


_INFO = plsc.get_sparse_core_info()
NC = _INFO.num_cores
NS = _INFO.num_subcores
NW = NC * NS
B_PER_W = N_TOK // NW
CHUNK = 40
NITER = B_PER_W // CHUNK
NBUF = 4
LAG = 2


def _gather_sc(table, idx):
    mesh = plsc.VectorSubcoreMesh(core_axis_name="c", subcore_axis_name="s")

    @functools.partial(
        pl.kernel,
        mesh=mesh,
        out_type=jax.ShapeDtypeStruct((N_TOK, DM), jnp.float32),
        scratch_types=[
            pltpu.VMEM((B_PER_W,), jnp.int32),
            pltpu.VMEM((NBUF, CHUNK, DM), jnp.float32),
        ]
        + [pltpu.SemaphoreType.DMA] * (2 * NBUF),
    )
    def k(table_hbm, idx_hbm, out_hbm, idx_v, rows_v, *sems):
        gsems, wsems = sems[:NBUF], sems[NBUF:]
        wid = lax.axis_index("s") * NC + lax.axis_index("c")
        base = wid * B_PER_W
        pltpu.sync_copy(idx_hbm.at[pl.ds(base, B_PER_W)], idx_v)

        def start_gather(i, buf):
            pltpu.async_copy(
                table_hbm.at[idx_v.at[pl.ds(i * CHUNK, CHUNK)]],
                rows_v.at[buf],
                gsems[buf],
            )

        def wait_gather(buf):
            pltpu.make_async_copy(
                table_hbm.at[idx_v.at[pl.ds(0, CHUNK)]],
                rows_v.at[buf],
                gsems[buf],
            ).wait()

        def start_write(i, buf):
            pltpu.async_copy(
                rows_v.at[buf],
                out_hbm.at[pl.ds(base + i * CHUNK, CHUNK)],
                wsems[buf],
            )

        def wait_write(buf):
            pltpu.make_async_copy(
                rows_v.at[buf],
                out_hbm.at[pl.ds(base, CHUNK)],
                wsems[buf],
            ).wait()

        for b in range(LAG):
            start_gather(b, b)

        def body(j, _):
            for b in range(NBUF):
                i = j * NBUF + b
                wait_gather(b)
                start_write(i, b)
                nxt = i + LAG

                @pl.when(nxt >= NBUF)
                def _():
                    wait_write((b + LAG) % NBUF)

                @pl.when(nxt < NITER)
                def _():
                    start_gather(nxt, (b + LAG) % NBUF)
            return 0

        lax.fori_loop(0, NITER // NBUF, body, 0)
        for b in range(LAG):
            wait_write((NITER - LAG + b) % NBUF)

    return k(table, idx)


def kernel(x, word_vectors, W):
    proj = _project_table(word_vectors, W)
    flat = _gather_sc(proj, x.reshape(-1))
    return flat.reshape(B, L, DM)


# speedup vs baseline: 1.7449x; 1.0248x over previous
import functools
import math

import jax
import jax.numpy as jnp
from jax import lax
from jax.experimental import pallas as pl
from jax.experimental.pallas import tpu as pltpu
from jax.experimental.pallas import tpu_sc as plsc

VOCAB = 100000
EMB = 300
DM = 512
B = 1024
L = 200
N_TOK = B * L
SCALE = math.sqrt(DM)


BM = 4096


def _proj_body(wvt_ref, w_ref, out_ref):
    out_ref[...] = lax.dot_general(
        wvt_ref[...], w_ref[...],
        dimension_numbers=(((0,), (1,)), ((), ())),
        preferred_element_type=jnp.float32,
    ) * SCALE


def _project_table(word_vectors, W):
    wvt = word_vectors.T
    return pl.pallas_call(
        _proj_body,
        grid=((VOCAB + BM - 1) // BM,),
        in_specs=[
            pl.BlockSpec((EMB, BM), lambda i: (0, i)),
            pl.BlockSpec((DM, EMB), lambda i: (0, 0)),
        ],
        out_specs=pl.BlockSpec((BM, DM), lambda i: (i, 0)),
        out_shape=jax.ShapeDtypeStruct((VOCAB, DM), jnp.float32),
    )(wvt, W)


_INFO = plsc.get_sparse_core_info()
NC = _INFO.num_cores
NS = _INFO.num_subcores
NW = NC * NS
B_PER_W = N_TOK // NW
CHUNK = 40
NITER = B_PER_W // CHUNK
NBUF = 4
LAG = 2


def _gather_sc(table, idx):
    mesh = plsc.VectorSubcoreMesh(core_axis_name="c", subcore_axis_name="s")

    @functools.partial(
        pl.kernel,
        mesh=mesh,
        out_type=jax.ShapeDtypeStruct((N_TOK, DM), jnp.float32),
        scratch_types=[
            pltpu.VMEM((B_PER_W,), jnp.int32),
            pltpu.VMEM((NBUF, CHUNK, DM), jnp.float32),
        ]
        + [pltpu.SemaphoreType.DMA] * (2 * NBUF),
    )
    def k(table_hbm, idx_hbm, out_hbm, idx_v, rows_v, *sems):
        gsems, wsems = sems[:NBUF], sems[NBUF:]
        wid = lax.axis_index("s") * NC + lax.axis_index("c")
        base = wid * B_PER_W
        pltpu.sync_copy(idx_hbm.at[pl.ds(base, B_PER_W)], idx_v)

        def start_gather(i, buf):
            pltpu.async_copy(
                table_hbm.at[idx_v.at[pl.ds(i * CHUNK, CHUNK)]],
                rows_v.at[buf],
                gsems[buf],
            )

        def wait_gather(buf):
            pltpu.make_async_copy(
                table_hbm.at[idx_v.at[pl.ds(0, CHUNK)]],
                rows_v.at[buf],
                gsems[buf],
            ).wait()

        def start_write(i, buf):
            pltpu.async_copy(
                rows_v.at[buf],
                out_hbm.at[pl.ds(base + i * CHUNK, CHUNK)],
                wsems[buf],
            )

        def wait_write(buf):
            pltpu.make_async_copy(
                rows_v.at[buf],
                out_hbm.at[pl.ds(base, CHUNK)],
                wsems[buf],
            ).wait()

        for b in range(LAG):
            start_gather(b, b)

        def body(j, _):
            for b in range(NBUF):
                i = j * NBUF + b
                wait_gather(b)
                start_write(i, b)
                nxt = i + LAG

                @pl.when(nxt >= NBUF)
                def _():
                    wait_write((b + LAG) % NBUF)

                @pl.when(nxt < NITER)
                def _():
                    start_gather(nxt, (b + LAG) % NBUF)
            return 0

        lax.fori_loop(0, NITER // NBUF, body, 0)
        for b in range(LAG):
            wait_write((NITER - LAG + b) % NBUF)

    return k(table, idx)


def kernel(x, word_vectors, W):
    proj = _project_table(word_vectors, W)
    flat = _gather_sc(proj, x.reshape(-1))
    return flat.reshape(B, L, DM)


# speedup vs baseline: 1.7468x; 1.0011x over previous
import functools
import math

import jax
import jax.numpy as jnp
from jax import lax
from jax.experimental import pallas as pl
from jax.experimental.pallas import tpu as pltpu
from jax.experimental.pallas import tpu_sc as plsc

VOCAB = 100000
EMB = 300
DM = 512
B = 1024
L = 200
N_TOK = B * L
SCALE = math.sqrt(DM)


BM = 4096


def _proj_body(wvt_ref, w_ref, out_ref):
    out_ref[...] = lax.dot_general(
        wvt_ref[...], w_ref[...],
        dimension_numbers=(((0,), (1,)), ((), ())),
        preferred_element_type=jnp.float32,
    ) * SCALE


def _project_table(word_vectors, W):
    wvt = word_vectors.T
    return pl.pallas_call(
        _proj_body,
        grid=((VOCAB + BM - 1) // BM,),
        in_specs=[
            pl.BlockSpec((EMB, BM), lambda i: (0, i)),
            pl.BlockSpec((DM, EMB), lambda i: (0, 0)),
        ],
        out_specs=pl.BlockSpec((BM, DM), lambda i: (i, 0)),
        out_shape=jax.ShapeDtypeStruct((VOCAB, DM), jnp.float32),
    )(wvt, W)


_INFO = plsc.get_sparse_core_info()
NC = _INFO.num_cores
NS = _INFO.num_subcores
NW = NC * NS
B_PER_W = N_TOK // NW
CHUNK = 80
NITER = B_PER_W // CHUNK
NBUF = 2


def _gather_sc(table, idx):
    mesh = plsc.VectorSubcoreMesh(core_axis_name="c", subcore_axis_name="s")

    @functools.partial(
        pl.kernel,
        mesh=mesh,
        out_type=jax.ShapeDtypeStruct((N_TOK, DM), jnp.float32),
        scratch_types=[
            pltpu.VMEM((B_PER_W,), jnp.int32),
            pltpu.VMEM((NBUF, CHUNK, DM), jnp.float32),
        ]
        + [pltpu.SemaphoreType.DMA] * NBUF,
    )
    def k(table_hbm, idx_hbm, out_hbm, idx_v, rows_v, *gsems):
        wid = lax.axis_index("s") * NC + lax.axis_index("c")
        base = wid * B_PER_W
        pltpu.sync_copy(idx_hbm.at[pl.ds(base, B_PER_W)], idx_v)

        def start_gather(i, buf):
            pltpu.async_copy(
                table_hbm.at[idx_v.at[pl.ds(i * CHUNK, CHUNK)]],
                rows_v.at[buf],
                gsems[buf],
            )

        def wait_gather(buf):
            pltpu.make_async_copy(
                table_hbm.at[idx_v.at[pl.ds(0, CHUNK)]],
                rows_v.at[buf],
                gsems[buf],
            ).wait()

        for b in range(NBUF):
            start_gather(b, b)

        def body(j, _):
            for b in range(NBUF):
                i = j * NBUF + b
                wait_gather(b)
                pltpu.sync_copy(
                    rows_v.at[b],
                    out_hbm.at[pl.ds(base + i * CHUNK, CHUNK)],
                )

                @pl.when(i + NBUF < NITER)
                def _():
                    start_gather(i + NBUF, b)
            return 0

        lax.fori_loop(0, NITER // NBUF, body, 0)

    return k(table, idx)


def kernel(x, word_vectors, W):
    proj = _project_table(word_vectors, W)
    flat = _gather_sc(proj, x.reshape(-1))
    return flat.reshape(B, L, DM)
